# Initial kernel scaffold; baseline (speedup 1.0000x reference)
import jax
import jax.numpy as jnp
from jax.experimental import pallas as pl


def kernel(x, edge_index, batch, W1r, b1r, W1o, W2r, b2r, W2o, W3r, b3r, W3o, W4r, b4r, W4o, W5r, b5r, W5o, lin1_W, lin1_b, lin2_W, lin2_b):
    raise NotImplementedError("write your pallas kernel here")


# speedup vs baseline: 3.7173x; 3.7173x over previous
import functools

import jax
import jax.numpy as jnp
from jax import lax
from jax.experimental import pallas as pl
from jax.experimental.pallas import tpu as pltpu
from jax.experimental.pallas import tpu_sc as plsc

N = 10000
E = 320000
NF = 14
DIM = 128
NG = 256

NW = 32
CHUNK = 128
ECH = 79
EP = NW * ECH * CHUNK
NT = 10240
ROWS_PER_TILE = NT // 16

PCH = 64
PNCH = 5
NODES_PER_W = PCH * PNCH


def _sc_segsum():
    mesh = plsc.VectorSubcoreMesh(core_axis_name="c", subcore_axis_name="s")

    @functools.partial(
        pl.kernel,
        out_type=jax.ShapeDtypeStruct((2 * NT, DIM), jnp.float32),
        mesh=mesh,
        scratch_types=[
            pltpu.VMEM((ECH, CHUNK), jnp.int32),
            pltpu.VMEM((ECH, CHUNK), jnp.int32),
            pltpu.VMEM((CHUNK, DIM), jnp.float32),
            pltpu.VMEM_SHARED((NT, DIM), jnp.float32),
            pltpu.SemaphoreType.DMA,
        ],
    )
    def k(m_hbm, srcp_hbm, dstp_hbm, zeros_hbm, out_hbm,
          src_v, dst_v, rows_v, agg_sh, sem):
        c = lax.axis_index("c")
        s = lax.axis_index("s")
        w = c * 16 + s
        pltpu.sync_copy(zeros_hbm, agg_sh.at[pl.ds(s * ROWS_PER_TILE, ROWS_PER_TILE)])
        pltpu.sync_copy(srcp_hbm.at[w], src_v)
        pltpu.sync_copy(dstp_hbm.at[w], dst_v)
        plsc.subcore_barrier()

        def body(g, _):
            pltpu.async_copy(m_hbm.at[src_v.at[g]], rows_v, sem).wait()
            pltpu.sync_copy(rows_v, agg_sh.at[dst_v.at[g]], add=True)
            return _

        lax.fori_loop(0, ECH, body, 0, unroll=False)
        plsc.subcore_barrier()
        pltpu.sync_copy(
            agg_sh.at[pl.ds(s * ROWS_PER_TILE, ROWS_PER_TILE)],
            out_hbm.at[pl.ds(c * NT + s * ROWS_PER_TILE, ROWS_PER_TILE)],
        )

    return k


def _sc_pool():
    mesh = plsc.VectorSubcoreMesh(core_axis_name="c", subcore_axis_name="s")

    @functools.partial(
        pl.kernel,
        out_type=jax.ShapeDtypeStruct((2 * NG, DIM), jnp.float32),
        mesh=mesh,
        scratch_types=[
            pltpu.VMEM((PNCH, PCH), jnp.int32),
            pltpu.VMEM((PCH, DIM), jnp.float32),
            pltpu.VMEM_SHARED((NG, DIM), jnp.float32),
        ],
    )
    def k(h_hbm, batchp_hbm, zeros_hbm, out_hbm, bidx_v, rows_v, pool_sh):
        c = lax.axis_index("c")
        s = lax.axis_index("s")
        w = c * 16 + s
        pltpu.sync_copy(zeros_hbm.at[pl.ds(0, NG // 16)],
                        pool_sh.at[pl.ds(s * (NG // 16), NG // 16)])
        pltpu.sync_copy(batchp_hbm.at[w], bidx_v)
        plsc.subcore_barrier()

        def body(j, _):
            pltpu.sync_copy(h_hbm.at[pl.ds(w * NODES_PER_W + j * PCH, PCH)], rows_v)
            pltpu.sync_copy(rows_v, pool_sh.at[bidx_v.at[j]], add=True)
            return _

        lax.fori_loop(0, PNCH, body, 0, unroll=False)
        plsc.subcore_barrier()
        pltpu.sync_copy(
            pool_sh.at[pl.ds(s * (NG // 16), NG // 16)],
            out_hbm.at[pl.ds(c * NG + s * (NG // 16), NG // 16)],
        )

    return k


RB = 1000


def _tc_first(x_ref, w1r_ref, w1o_ref, b1_ref, m_ref, r_ref):
    x = x_ref[...]
    m_ref[...] = lax.dot_general(x, w1r_ref[...], (((1,), (1,)), ((), ())),
                                 preferred_element_type=jnp.float32)
    r_ref[...] = lax.dot_general(x, w1o_ref[...], (((1,), (1,)), ((), ())),
                                 preferred_element_type=jnp.float32) + b1_ref[...]


def _tc_mid(a0_ref, a1_ref, root_ref, wr_ref, wo_ref, b_ref, m_ref, r_ref):
    h = jnp.maximum(a0_ref[...] + a1_ref[...] + root_ref[...], 0.0)
    m_ref[...] = lax.dot_general(h, wr_ref[...], (((1,), (1,)), ((), ())),
                                 preferred_element_type=jnp.float32)
    r_ref[...] = lax.dot_general(h, wo_ref[...], (((1,), (1,)), ((), ())),
                                 preferred_element_type=jnp.float32) + b_ref[...]


def _tc_last(a0_ref, a1_ref, root_ref, h_ref):
    h_ref[...] = jnp.maximum(a0_ref[...] + a1_ref[...] + root_ref[...], 0.0)


def _tc_head(p0_ref, p1_ref, w1_ref, b1_ref, w2_ref, b2_ref, out_ref):
    p = p0_ref[...] + p1_ref[...]
    h2 = jnp.maximum(
        lax.dot_general(p, w1_ref[...], (((1,), (1,)), ((), ())),
                        preferred_element_type=jnp.float32) + b1_ref[...], 0.0)
    lg = lax.dot_general(h2, w2_ref[...], (((1,), (1,)), ((), ())),
                         preferred_element_type=jnp.float32) + b2_ref[...]
    col = lax.broadcasted_iota(jnp.int32, lg.shape, 1)
    lg = jnp.where(col < 2, lg, -1e30)
    mx = jnp.max(lg, axis=1, keepdims=True)
    lse = mx + jnp.log(jnp.sum(jnp.exp(lg - mx), axis=1, keepdims=True))
    out_ref[...] = lg - lse


def _row_grid(nrows):
    return nrows // RB


def kernel(x, edge_index, batch, W1r, b1r, W1o, W2r, b2r, W2o, W3r, b3r, W3o,
           W4r, b4r, W4o, W5r, b5r, W5o, lin1_W, lin1_b, lin2_W, lin2_b):
    f32 = jnp.float32
    src = edge_index[0]
    dst = edge_index[1]
    pad = EP - E
    srcp = jnp.concatenate([src, jnp.zeros((pad,), jnp.int32)]).reshape(NW, ECH, CHUNK)
    dstp = jnp.concatenate([dst, jnp.full((pad,), N, jnp.int32)]).reshape(NW, ECH, CHUNK)
    zeros_tile = jnp.zeros((ROWS_PER_TILE, DIM), f32)

    batchp = jnp.concatenate([batch, jnp.zeros((NT - N,), jnp.int32)]).reshape(NW, PNCH, PCH)

    sc_seg = _sc_segsum()
    sc_pool = _sc_pool()

    grid = _row_grid(N)
    row_in = pl.BlockSpec((RB, DIM), lambda i: (i, 0))
    full_w = pl.BlockSpec((DIM, DIM), lambda i: (0, 0))
    full_b = pl.BlockSpec((1, DIM), lambda i: (0, 0))

    first = pl.pallas_call(
        _tc_first,
        grid=(grid,),
        in_specs=[pl.BlockSpec((RB, NF), lambda i: (i, 0)),
                  pl.BlockSpec((DIM, NF), lambda i: (0, 0)),
                  pl.BlockSpec((DIM, NF), lambda i: (0, 0)),
                  full_b],
        out_specs=[row_in, row_in],
        out_shape=[jax.ShapeDtypeStruct((N, DIM), f32)] * 2,
    )
    mid = pl.pallas_call(
        _tc_mid,
        grid=(grid,),
        in_specs=[row_in, row_in, row_in, full_w, full_w, full_b],
        out_specs=[row_in, row_in],
        out_shape=[jax.ShapeDtypeStruct((N, DIM), f32)] * 2,
    )
    last = pl.pallas_call(
        _tc_last,
        grid=(grid,),
        in_specs=[row_in, row_in, row_in],
        out_specs=row_in,
        out_shape=jax.ShapeDtypeStruct((N, DIM), f32),
    )
    head = pl.pallas_call(
        _tc_head,
        in_specs=[pl.BlockSpec((NG, DIM), lambda: (0, 0))] * 2
        + [pl.BlockSpec((DIM, DIM), lambda: (0, 0)),
           pl.BlockSpec((1, DIM), lambda: (0, 0)),
           pl.BlockSpec((DIM, DIM), lambda: (0, 0)),
           pl.BlockSpec((1, DIM), lambda: (0, 0))],
        out_specs=pl.BlockSpec((NG, DIM), lambda: (0, 0)),
        out_shape=jax.ShapeDtypeStruct((NG, DIM), f32),
    )

    b1 = b1r.reshape(1, DIM)

    m, root = first(x, W1r, W1o, b1)
    Wrs = [W2r, W3r, W4r, W5r]
    Wos = [W2o, W3o, W4o, W5o]
    bs = [b2r, b3r, b4r, b5r]
    for i in range(4):
        agg = sc_seg(m, srcp, dstp, zeros_tile)
        a0 = agg[:NT]
        a1 = agg[NT:]
        m, root = mid(a0, a1, root, Wrs[i], Wos[i], bs[i].reshape(1, DIM))
    agg = sc_seg(m, srcp, dstp, zeros_tile)
    h5 = last(agg[:NT], agg[NT:], root)

    h5p = jnp.concatenate([h5, jnp.zeros((NT - N, DIM), f32)], axis=0)
    pooled = sc_pool(h5p, batchp, zeros_tile)

    lin2_Wp = jnp.concatenate([lin2_W, jnp.zeros((DIM - 2, DIM), f32)], axis=0)
    lin2_bp = jnp.concatenate([lin2_b, jnp.zeros((DIM - 2,), f32)]).reshape(1, DIM)
    out = head(pooled[:NG], pooled[NG:], lin1_W, lin1_b.reshape(1, DIM),
               lin2_Wp, lin2_bp)
    return out[:, :2]
